# initial kernel scaffold (unmeasured)
import jax
import jax.numpy as jnp
from jax import lax
from jax.experimental import pallas as pl
from jax.experimental.pallas import tpu as pltpu


def kernel(
    x,
):
    def body(*refs):
        pass

    out_shape = jax.ShapeDtypeStruct(..., jnp.float32)
    return pl.pallas_call(body, out_shape=out_shape)(...)


# baseline (device time: 86323 ns/iter reference)
import jax
import jax.numpy as jnp
from jax import lax
from jax.experimental import pallas as pl
from jax.experimental.pallas import tpu as pltpu

N_DEV = 4


def kernel(x):
    m, n = x.shape[1], x.shape[2]
    x2 = x.reshape(m, n)
    ch = m // N_DEV

    def body(x_ref, out_ref, rbuf, send_sems, recv_sems):
        i = lax.axis_index("i")
        right = (i + 1) % N_DEV
        left = (i - 1) % N_DEV

        barrier = pltpu.get_barrier_semaphore()
        for nbr in (left, right):
            pl.semaphore_signal(
                barrier, inc=1,
                device_id=(nbr,), device_id_type=pl.DeviceIdType.MESH,
            )
        pl.semaphore_wait(barrier, 2)

        out_ref[...] = x_ref[...].astype(out_ref.dtype)

        def chunk(ref, c):
            return ref.at[pl.ds(c * ch, ch), :]

        pending = []

        for s in range(N_DEV - 1):
            send_c = (i - s) % N_DEV
            recv_c = (i - s - 1) % N_DEV
            rdma = pltpu.make_async_remote_copy(
                src_ref=chunk(out_ref, send_c),
                dst_ref=rbuf.at[s],
                send_sem=send_sems.at[s],
                recv_sem=recv_sems.at[s],
                device_id=(right,),
                device_id_type=pl.DeviceIdType.MESH,
            )
            rdma.start()
            rdma.wait_recv()
            acc = chunk(out_ref, recv_c)
            acc[...] = acc[...] + rbuf[s]
            pending.append(rdma)

        for s in range(N_DEV - 1):
            send_c = (i + 1 - s) % N_DEV
            rdma = pltpu.make_async_remote_copy(
                src_ref=chunk(out_ref, send_c),
                dst_ref=chunk(out_ref, send_c),
                send_sem=send_sems.at[N_DEV - 1 + s],
                recv_sem=recv_sems.at[N_DEV - 1 + s],
                device_id=(right,),
                device_id_type=pl.DeviceIdType.MESH,
            )
            rdma.start()
            rdma.wait_recv()
            pending.append(rdma)

        for r in pending:
            r.wait_send()

    return pl.pallas_call(
        body,
        out_shape=jax.ShapeDtypeStruct((m, n), jnp.bfloat16),
        in_specs=[pl.BlockSpec(memory_space=pltpu.VMEM)],
        out_specs=pl.BlockSpec(memory_space=pltpu.VMEM),
        scratch_shapes=[
            pltpu.VMEM((N_DEV - 1, ch, n), jnp.bfloat16),
            pltpu.SemaphoreType.DMA((2 * (N_DEV - 1),)),
            pltpu.SemaphoreType.DMA((2 * (N_DEV - 1),)),
        ],
        compiler_params=pltpu.CompilerParams(collective_id=0),
    )(x2)


# device time: 52773 ns/iter; 1.6357x vs baseline; 1.6357x over previous
import jax
import jax.numpy as jnp
from jax import lax
from jax.experimental import pallas as pl
from jax.experimental.pallas import tpu as pltpu

N_DEV = 4
N_DIR = 2


def kernel(x):
    m, n = x.shape[1], x.shape[2]
    x2 = x.reshape(m, n)
    half = m // N_DIR
    ch = half // N_DEV

    def body(x_ref, out_ref, rbuf, send_sems, recv_sems):
        i = lax.axis_index("i")
        right = (i + 1) % N_DEV
        left = (i - 1) % N_DEV
        nbr = (right, left)

        barrier = pltpu.get_barrier_semaphore()
        for b in (left, right):
            pl.semaphore_signal(
                barrier, inc=1,
                device_id=(b,), device_id_type=pl.DeviceIdType.MESH,
            )
        pl.semaphore_wait(barrier, 2)

        out_ref[...] = x_ref[...].astype(out_ref.dtype)

        def chunk(d, c):
            return out_ref.at[pl.ds(d * half + c * ch, ch), :]

        pending = []

        for s in range(N_DEV - 1):
            started = []
            for d in range(N_DIR):
                send_c = (i - s) % N_DEV if d == 0 else (i + s) % N_DEV
                rdma = pltpu.make_async_remote_copy(
                    src_ref=chunk(d, send_c),
                    dst_ref=rbuf.at[d, s],
                    send_sem=send_sems.at[d, s],
                    recv_sem=recv_sems.at[d, s],
                    device_id=(nbr[d],),
                    device_id_type=pl.DeviceIdType.MESH,
                )
                rdma.start()
                started.append(rdma)
            for d in range(N_DIR):
                recv_c = (i - s - 1) % N_DEV if d == 0 else (i + s + 1) % N_DEV
                started[d].wait_recv()
                acc = chunk(d, recv_c)
                acc[...] = acc[...] + rbuf[d, s]
                pending.append(started[d])

        for s in range(N_DEV - 1):
            started = []
            for d in range(N_DIR):
                send_c = (i + 1 - s) % N_DEV if d == 0 else (i + 3 + s) % N_DEV
                rdma = pltpu.make_async_remote_copy(
                    src_ref=chunk(d, send_c),
                    dst_ref=chunk(d, send_c),
                    send_sem=send_sems.at[d, N_DEV - 1 + s],
                    recv_sem=recv_sems.at[d, N_DEV - 1 + s],
                    device_id=(nbr[d],),
                    device_id_type=pl.DeviceIdType.MESH,
                )
                rdma.start()
                started.append(rdma)
            for d in range(N_DIR):
                started[d].wait_recv()
                pending.append(started[d])

        for r in pending:
            r.wait_send()

    return pl.pallas_call(
        body,
        out_shape=jax.ShapeDtypeStruct((m, n), jnp.bfloat16),
        in_specs=[pl.BlockSpec(memory_space=pltpu.VMEM)],
        out_specs=pl.BlockSpec(memory_space=pltpu.VMEM),
        scratch_shapes=[
            pltpu.VMEM((N_DIR, N_DEV - 1, ch, n), jnp.bfloat16),
            pltpu.SemaphoreType.DMA((N_DIR, 2 * (N_DEV - 1))),
            pltpu.SemaphoreType.DMA((N_DIR, 2 * (N_DEV - 1))),
        ],
        compiler_params=pltpu.CompilerParams(collective_id=0),
    )(x2)


# device time: 43278 ns/iter; 1.9946x vs baseline; 1.2194x over previous
import jax
import jax.numpy as jnp
from jax import lax
from jax.experimental import pallas as pl
from jax.experimental.pallas import tpu as pltpu

N_DEV = 4
N_DIR = 2
K_SUB = 2


def kernel(x):
    m, n = x.shape[1], x.shape[2]
    x2 = x.reshape(m, n)
    half = m // N_DIR
    ch = half // N_DEV
    sub = ch // K_SUB

    def body(x_ref, out_ref, rbuf, send_sems, recv_sems):
        i = lax.axis_index("i")
        right = (i + 1) % N_DEV
        left = (i - 1) % N_DEV
        nbr = (right, left)

        barrier = pltpu.get_barrier_semaphore()
        for b in (left, right):
            pl.semaphore_signal(
                barrier, inc=1,
                device_id=(b,), device_id_type=pl.DeviceIdType.MESH,
            )
        pl.semaphore_wait(barrier, 2)

        descs = {}

        def sub_slice(d, c, j):
            return pl.ds(d * half + c * ch + j * sub, sub)

        def start(d, s, j, c, to_rbuf):
            src = out_ref.at[sub_slice(d, c, j), :]
            dst = rbuf.at[d, s, j] if to_rbuf else src
            rdma = pltpu.make_async_remote_copy(
                src_ref=src,
                dst_ref=dst,
                send_sem=send_sems.at[d, s, j],
                recv_sem=recv_sems.at[d, s, j],
                device_id=(nbr[d],),
                device_id_type=pl.DeviceIdType.MESH,
            )
            rdma.start()
            descs[(d, s, j)] = rdma

        for c_off in range(N_DEV):
            c = (i + c_off) % N_DEV
            for d in range(N_DIR):
                rows = pl.ds(d * half + c * ch, ch)
                out_ref[rows, :] = x_ref[rows, :].astype(out_ref.dtype)
            if c_off == 0:
                for j in range(K_SUB):
                    for d in range(N_DIR):
                        start(d, 0, j, i, to_rbuf=True)

        for s in range(1, N_DEV - 1):
            for j in range(K_SUB):
                for d in range(N_DIR):
                    descs[(d, s - 1, j)].wait_recv()
                    c = (i - s) % N_DEV if d == 0 else (i + s) % N_DEV
                    acc = out_ref.at[sub_slice(d, c, j), :]
                    acc[...] = acc[...] + rbuf[d, s - 1, j]
                    start(d, s, j, c, to_rbuf=True)

        for j in range(K_SUB):
            for d in range(N_DIR):
                descs[(d, N_DEV - 2, j)].wait_recv()
                c = (i + 1) % N_DEV if d == 0 else (i + 3) % N_DEV
                acc = out_ref.at[sub_slice(d, c, j), :]
                acc[...] = acc[...] + rbuf[d, N_DEV - 2, j]
                start(d, 3, j, c, to_rbuf=False)

        for s in range(1, N_DEV - 1):
            for j in range(K_SUB):
                for d in range(N_DIR):
                    descs[(d, 3 + s - 1, j)].wait_recv()
                    c = (i + 1 - s) % N_DEV if d == 0 else (i + 3 + s) % N_DEV
                    start(d, 3 + s, j, c, to_rbuf=False)

        for j in range(K_SUB):
            for d in range(N_DIR):
                descs[(d, 5, j)].wait_recv()
        for r in descs.values():
            r.wait_send()

    return pl.pallas_call(
        body,
        out_shape=jax.ShapeDtypeStruct((m, n), jnp.bfloat16),
        in_specs=[pl.BlockSpec(memory_space=pltpu.VMEM)],
        out_specs=pl.BlockSpec(memory_space=pltpu.VMEM),
        scratch_shapes=[
            pltpu.VMEM((N_DIR, N_DEV - 1, K_SUB, sub, n), jnp.bfloat16),
            pltpu.SemaphoreType.DMA((N_DIR, 2 * (N_DEV - 1), K_SUB)),
            pltpu.SemaphoreType.DMA((N_DIR, 2 * (N_DEV - 1), K_SUB)),
        ],
        compiler_params=pltpu.CompilerParams(collective_id=0),
    )(x2)
